# baseline (speedup 1.0000x reference)
import functools

import jax
import jax.numpy as jnp
from jax import lax
from jax.experimental import pallas as pl
from jax.experimental.pallas import tpu as pltpu
from jax.experimental.pallas import tpu_sc as plsc

EPS = 1e-7
MIN_NORM = 1e-15

N_NODES = 10000
N_EDGES = 320000
D = 129
DP = 144

NC = 2
NS = 16
LANES = 16
NW = NC * NS

BLK = 112
NB = 90
E_PAD = NW * NB * BLK

N_ACC = 10240
ROWS_PER_TILE = N_ACC // NS
ZCHUNK = 80

TC_BLK = 1000


def _arcosh(x):
    return jnp.log(x + jnp.sqrt(jnp.maximum(x * x - 1.0, 0.0)))


_BCAST_DNUMS = lax.GatherDimensionNumbers(
    offset_dims=(), collapsed_slice_dims=(0,), start_index_map=(0,))


def _bcast_lane(v, lane):
    idx = jnp.full((LANES, 1), lane, jnp.int32)
    return lax.gather(v, idx, _BCAST_DNUMS, (1,),
                      mode=lax.GatherScatterMode.PROMISE_IN_BOUNDS)


def _sinh(v):
    e = jnp.exp(v)
    return 0.5 * (e - 1.0 / e)


def _linear_body(x_ref, w_ref, h_ref):
    x0 = x_ref[:, 0:1]
    y = x_ref[:, 1:D]
    y_norm = jnp.maximum(
        jnp.sqrt(jnp.sum(y * y, axis=1, keepdims=True)), MIN_NORM)
    theta = jnp.maximum(x0, 1.0 + EPS)
    s = y * (_arcosh(theta) / y_norm)
    mu = lax.dot_general(s, w_ref[...], (((1,), (1,)), ((), ())),
                         preferred_element_type=jnp.float32)
    v_norm = jnp.maximum(
        jnp.sqrt(jnp.sum(mu * mu, axis=1, keepdims=True)), MIN_NORM)
    sp = _sinh(v_norm) * mu / v_norm
    t = jnp.sqrt(jnp.maximum(
        1.0 + jnp.sum(sp * sp, axis=1, keepdims=True), EPS))
    pad = jnp.zeros((sp.shape[0], DP - D), jnp.float32)
    h_ref[...] = jnp.concatenate([t, sp, pad], axis=1)


def _lorentz_linear(x, weight):
    grid = N_NODES // TC_BLK
    return pl.pallas_call(
        _linear_body,
        grid=(grid,),
        in_specs=[
            pl.BlockSpec((TC_BLK, D), lambda i: (i, 0)),
            pl.BlockSpec((128, 128), lambda i: (0, 0)),
        ],
        out_specs=pl.BlockSpec((TC_BLK, DP), lambda i: (i, 0)),
        out_shape=jax.ShapeDtypeStruct((N_NODES, DP), jnp.float32),
    )(x, weight)


def _scale_rows(rows_ref, w_ref, y):
    def _grp(g, _):
        wv = w_ref[y, pl.ds(g * LANES, LANES)]
        for e in range(LANES):
            wvec = _bcast_lane(wv, e)
            row = g * LANES + e
            for j in range(DP // LANES):
                sl = pl.ds(j * LANES, LANES)
                rows_ref[row, sl] = rows_ref[row, sl] * wvec
        return 0
    lax.fori_loop(0, BLK // LANES, _grp, 0)


def _sc_body(h_hbm, src_hbm, dst_hbm, w_hbm, out_hbm,
             sidx_r, didx_r, w_r, rows0, rows1, acc_sh,
             sg0, sg1, ss0, ss1, si0, si1, si2, si3):
    c = lax.axis_index("c")
    s = lax.axis_index("s")
    wid = c * NS + s
    last = jnp.int32(NB - 1)

    bufs = (rows0, rows1)
    gsems = (sg0, sg1)
    ssems = (ss0, ss1)
    isems = (si0, si1, si2, si3)

    def _start_idx(b, y):
        base = (wid * NB + b) * BLK
        pltpu.async_copy(src_hbm.at[pl.ds(base, BLK)], sidx_r.at[y],
                         isems[y])
        pltpu.async_copy(dst_hbm.at[pl.ds(base, BLK)], didx_r.at[y],
                         isems[y])
        pltpu.async_copy(w_hbm.at[pl.ds(base, BLK)], w_r.at[y], isems[y])

    def _wait_idx(y):
        pltpu.make_async_copy(src_hbm.at[pl.ds(0, BLK)], sidx_r.at[y],
                              isems[y]).wait()
        pltpu.make_async_copy(dst_hbm.at[pl.ds(0, BLK)], didx_r.at[y],
                              isems[y]).wait()
        pltpu.make_async_copy(w_hbm.at[pl.ds(0, BLK)], w_r.at[y],
                              isems[y]).wait()

    def _start_g(x, y):
        pltpu.async_copy(h_hbm.at[sidx_r.at[y]], bufs[x], gsems[x])

    def _wait_g(x, y):
        pltpu.make_async_copy(h_hbm.at[sidx_r.at[y]], bufs[x],
                              gsems[x]).wait()

    def _start_s(x, y):
        pltpu.async_copy(bufs[x], acc_sh.at[didx_r.at[y]], ssems[x],
                         add=True)

    def _wait_s(x, y):
        pltpu.make_async_copy(bufs[x], acc_sh.at[didx_r.at[y]],
                              ssems[x]).wait()

    for y in range(4):
        _start_idx(jnp.int32(y), y)
    _wait_idx(0)
    _wait_idx(1)

    def _zero_row(i, _):
        for j in range(DP // LANES):
            rows0[i, pl.ds(j * LANES, LANES)] = jnp.zeros((LANES,),
                                                          jnp.float32)
        return 0
    lax.fori_loop(0, ZCHUNK, _zero_row, 0)
    for k in range(ROWS_PER_TILE // ZCHUNK):
        base = s * ROWS_PER_TILE + k * ZCHUNK
        pltpu.sync_copy(rows0.at[pl.ds(0, ZCHUNK)],
                        acc_sh.at[pl.ds(base, ZCHUNK)])
    plsc.subcore_barrier()

    _start_g(0, 0)
    _start_g(1, 1)

    _wait_g(0, 0)
    _scale_rows(rows0, w_r, 0)
    _start_s(0, 0)
    _wait_g(1, 1)
    _wait_s(0, 0)
    _start_idx(jnp.int32(4), 0)
    _wait_idx(2)
    _start_g(0, 2)
    _scale_rows(rows1, w_r, 1)
    _start_s(1, 1)

    def _quad(p, _):
        b = 2 + p * 4
        for k in range(4):
            x = k % 2
            y = (2 + k) % 4
            ynext = (3 + k) % 4
            yold = (1 + k) % 4
            _wait_g(x, y)
            _wait_s(1 - x, yold)
            _start_idx(jnp.minimum(b + k + 3, last), yold)
            _wait_idx(ynext)
            _start_g(1 - x, ynext)
            _scale_rows(bufs[x], w_r, y)
            _start_s(x, y)
        return 0
    lax.fori_loop(0, (NB - 2) // 4, _quad, 0)

    _wait_s(1, 1)
    _wait_g(0, 2)
    _wait_idx(3)
    _wait_idx(0)
    plsc.subcore_barrier()

    for k in range(ROWS_PER_TILE // ZCHUNK):
        base = s * ROWS_PER_TILE + k * ZCHUNK
        pltpu.sync_copy(acc_sh.at[pl.ds(base, ZCHUNK)],
                        out_hbm.at[c, pl.ds(base, ZCHUNK)])


def _sc_segment_sum(h_pad, src, dst, w):
    mesh = plsc.VectorSubcoreMesh(core_axis_name="c", subcore_axis_name="s",
                                  num_cores=NC, num_subcores=NS)
    kern = pl.kernel(
        _sc_body,
        out_type=jax.ShapeDtypeStruct((NC, N_ACC, DP), jnp.float32),
        mesh=mesh,
        compiler_params=pltpu.CompilerParams(use_tc_tiling_on_sc=False),
        scratch_types=[
            pltpu.VMEM((4, BLK), jnp.int32),
            pltpu.VMEM((4, BLK), jnp.int32),
            pltpu.VMEM((4, BLK), jnp.float32),
            pltpu.VMEM((BLK, DP), jnp.float32),
            pltpu.VMEM((BLK, DP), jnp.float32),
            pltpu.VMEM_SHARED((N_ACC, DP), jnp.float32),
            pltpu.SemaphoreType.DMA,
            pltpu.SemaphoreType.DMA,
            pltpu.SemaphoreType.DMA,
            pltpu.SemaphoreType.DMA,
            pltpu.SemaphoreType.DMA,
            pltpu.SemaphoreType.DMA,
            pltpu.SemaphoreType.DMA,
            pltpu.SemaphoreType.DMA,
        ],
    )
    return kern(h_pad, src, dst, w)


def _post_body(p_ref, o_ref):
    sx = p_ref[0] + p_ref[1]
    s0 = sx[:, 0:1]
    mdot = jnp.sum(sx * sx, axis=1, keepdims=True) - 2.0 * s0 * s0
    coef = 1.0 / jnp.sqrt(jnp.maximum(jnp.abs(mdot), EPS))
    h = coef * sx
    h0 = h[:, 0:1]
    y = h[:, 1:D]
    y_norm = jnp.maximum(
        jnp.sqrt(jnp.sum(y * y, axis=1, keepdims=True)), MIN_NORM)
    theta = jnp.maximum(h0, 1.0 + EPS)
    xt = jnp.maximum(_arcosh(theta) * y / y_norm, 0.0)
    v_norm = jnp.maximum(
        jnp.sqrt(jnp.sum(xt * xt, axis=1, keepdims=True)), MIN_NORM)
    sp = _sinh(v_norm) * xt / v_norm
    t = jnp.sqrt(jnp.maximum(
        1.0 + jnp.sum(sp * sp, axis=1, keepdims=True), EPS))
    o_ref[...] = sp / (t + 1.0)


def _postprocess(partials):
    grid = N_NODES // TC_BLK
    return pl.pallas_call(
        _post_body,
        grid=(grid,),
        in_specs=[pl.BlockSpec((NC, TC_BLK, DP), lambda i: (0, i, 0))],
        out_specs=pl.BlockSpec((TC_BLK, 128), lambda i: (i, 0)),
        out_shape=jax.ShapeDtypeStruct((N_NODES, 128), jnp.float32),
    )(partials)


def kernel(x, edge_index, edge_weight, weight, dec_bias):
    h_pad = _lorentz_linear(x, weight)

    pad = E_PAD - N_EDGES
    pad_idx = (jnp.arange(pad, dtype=jnp.int32) * 97) % N_NODES
    src = jnp.concatenate([edge_index[0], pad_idx])
    dst = jnp.concatenate([edge_index[1], pad_idx])
    w = jnp.concatenate([edge_weight, jnp.zeros((pad,), jnp.float32)])

    partials = _sc_segment_sum(h_pad, src, dst, w)
    return _postprocess(partials)


# speedup vs baseline: 1.0527x; 1.0527x over previous
import functools

import jax
import jax.numpy as jnp
from jax import lax
from jax.experimental import pallas as pl
from jax.experimental.pallas import tpu as pltpu
from jax.experimental.pallas import tpu_sc as plsc

EPS = 1e-7
MIN_NORM = 1e-15

N_NODES = 10000
N_EDGES = 320000
D = 129
DP = 144

NC = 2
NS = 16
LANES = 16
NW = NC * NS

BLK = 128
NB = 80
NBLK_REAL = N_EDGES // BLK

N_ACC = 10240
ROWS_PER_TILE = N_ACC // NS
ZCHUNK = 80

TC_BLK = 1000


def _arcosh(x):
    return jnp.log(x + jnp.sqrt(jnp.maximum(x * x - 1.0, 0.0)))


_BCAST_DNUMS = lax.GatherDimensionNumbers(
    offset_dims=(), collapsed_slice_dims=(0,), start_index_map=(0,))


def _bcast_lane(v, lane):
    idx = jnp.full((LANES, 1), lane, jnp.int32)
    return lax.gather(v, idx, _BCAST_DNUMS, (1,),
                      mode=lax.GatherScatterMode.PROMISE_IN_BOUNDS)


def _sinh(v):
    e = jnp.exp(v)
    return 0.5 * (e - 1.0 / e)


def _linear_body(x_ref, w_ref, h_ref):
    x0 = x_ref[:, 0:1]
    y = x_ref[:, 1:D]
    y_norm = jnp.maximum(
        jnp.sqrt(jnp.sum(y * y, axis=1, keepdims=True)), MIN_NORM)
    theta = jnp.maximum(x0, 1.0 + EPS)
    s = y * (_arcosh(theta) / y_norm)
    mu = lax.dot_general(s, w_ref[...], (((1,), (1,)), ((), ())),
                         preferred_element_type=jnp.float32)
    v_norm = jnp.maximum(
        jnp.sqrt(jnp.sum(mu * mu, axis=1, keepdims=True)), MIN_NORM)
    sp = _sinh(v_norm) * mu / v_norm
    t = jnp.sqrt(jnp.maximum(
        1.0 + jnp.sum(sp * sp, axis=1, keepdims=True), EPS))
    pad = jnp.zeros((sp.shape[0], DP - D), jnp.float32)
    h_ref[...] = jnp.concatenate([t, sp, pad], axis=1)


def _lorentz_linear(x, weight):
    grid = N_NODES // TC_BLK
    return pl.pallas_call(
        _linear_body,
        grid=(grid,),
        in_specs=[
            pl.BlockSpec((TC_BLK, D), lambda i: (i, 0)),
            pl.BlockSpec((128, 128), lambda i: (0, 0)),
        ],
        out_specs=pl.BlockSpec((TC_BLK, DP), lambda i: (i, 0)),
        out_shape=jax.ShapeDtypeStruct((N_NODES, DP), jnp.float32),
    )(x, weight)


def _scale_rows(rows_ref, w_ref, y):
    def _grp(g, _):
        wv = w_ref[y, pl.ds(g * LANES, LANES)]
        for e in range(LANES):
            wvec = _bcast_lane(wv, e)
            row = g * LANES + e
            for j in range(DP // LANES):
                sl = pl.ds(j * LANES, LANES)
                rows_ref[row, sl] = rows_ref[row, sl] * wvec
        return 0
    lax.fori_loop(0, BLK // LANES, _grp, 0)


def _sc_body(h_hbm, ei_hbm, w_hbm, out_hbm,
             sidx_r, didx_r, w_r, rows0, rows1, acc_sh,
             sg0, sg1, ss0, ss1, si0, si1, si2, si3):
    c = lax.axis_index("c")
    s = lax.axis_index("s")
    wid = c * NS + s
    last = jnp.int32(NB - 1)

    bufs = (rows0, rows1)
    gsems = (sg0, sg1)
    ssems = (ss0, ss1)
    isems = (si0, si1, si2, si3)

    def _start_idx(b, y):
        g = wid * NB + b
        base = lax.rem(g, NBLK_REAL) * BLK
        pltpu.async_copy(ei_hbm.at[0, pl.ds(base, BLK)], sidx_r.at[y],
                         isems[y])
        pltpu.async_copy(ei_hbm.at[1, pl.ds(base, BLK)], didx_r.at[y],
                         isems[y])
        pltpu.async_copy(w_hbm.at[pl.ds(base, BLK)], w_r.at[y], isems[y])

    def _wait_idx(y):
        pltpu.make_async_copy(ei_hbm.at[0, pl.ds(0, BLK)], sidx_r.at[y],
                              isems[y]).wait()
        pltpu.make_async_copy(ei_hbm.at[1, pl.ds(0, BLK)], didx_r.at[y],
                              isems[y]).wait()
        pltpu.make_async_copy(w_hbm.at[pl.ds(0, BLK)], w_r.at[y],
                              isems[y]).wait()

    def _mask_pad_w(b, y):
        @pl.when(wid * NB + b >= NBLK_REAL)
        def _():
            for j in range(BLK // LANES):
                w_r[y, pl.ds(j * LANES, LANES)] = jnp.zeros((LANES,),
                                                            jnp.float32)

    def _start_g(x, y):
        pltpu.async_copy(h_hbm.at[sidx_r.at[y]], bufs[x], gsems[x])

    def _wait_g(x, y):
        pltpu.make_async_copy(h_hbm.at[sidx_r.at[y]], bufs[x],
                              gsems[x]).wait()

    def _start_s(x, y):
        pltpu.async_copy(bufs[x], acc_sh.at[didx_r.at[y]], ssems[x],
                         add=True)

    def _wait_s(x, y):
        pltpu.make_async_copy(bufs[x], acc_sh.at[didx_r.at[y]],
                              ssems[x]).wait()

    for y in range(4):
        _start_idx(jnp.int32(y), y)
    _wait_idx(0)
    _wait_idx(1)

    def _zero_row(i, _):
        for j in range(DP // LANES):
            rows0[i, pl.ds(j * LANES, LANES)] = jnp.zeros((LANES,),
                                                          jnp.float32)
        return 0
    lax.fori_loop(0, ZCHUNK, _zero_row, 0)
    for k in range(ROWS_PER_TILE // ZCHUNK):
        base = s * ROWS_PER_TILE + k * ZCHUNK
        pltpu.sync_copy(rows0.at[pl.ds(0, ZCHUNK)],
                        acc_sh.at[pl.ds(base, ZCHUNK)])
    plsc.subcore_barrier()

    _start_g(0, 0)
    _start_g(1, 1)

    _wait_g(0, 0)
    _scale_rows(rows0, w_r, 0)
    _start_s(0, 0)
    _wait_g(1, 1)
    _wait_s(0, 0)
    _start_idx(jnp.int32(4), 0)
    _wait_idx(2)
    _start_g(0, 2)
    _scale_rows(rows1, w_r, 1)
    _start_s(1, 1)

    def _quad(p, _):
        b = 2 + p * 4
        for k in range(4):
            x = k % 2
            y = (2 + k) % 4
            ynext = (3 + k) % 4
            yold = (1 + k) % 4
            _wait_g(x, y)
            _wait_s(1 - x, yold)
            _start_idx(jnp.minimum(b + k + 3, last), yold)
            _wait_idx(ynext)
            _start_g(1 - x, ynext)
            _mask_pad_w(b + k, y)
            _scale_rows(bufs[x], w_r, y)
            _start_s(x, y)
        return 0
    lax.fori_loop(0, (NB - 2) // 4, _quad, 0)

    _wait_s(1, 1)
    _wait_g(0, 2)
    _wait_idx(3)
    _wait_idx(0)
    plsc.subcore_barrier()

    for k in range(ROWS_PER_TILE // ZCHUNK):
        base = s * ROWS_PER_TILE + k * ZCHUNK
        pltpu.sync_copy(acc_sh.at[pl.ds(base, ZCHUNK)],
                        out_hbm.at[c, pl.ds(base, ZCHUNK)])


def _sc_segment_sum(h_pad, edge_index, edge_weight):
    mesh = plsc.VectorSubcoreMesh(core_axis_name="c", subcore_axis_name="s",
                                  num_cores=NC, num_subcores=NS)
    kern = pl.kernel(
        _sc_body,
        out_type=jax.ShapeDtypeStruct((NC, N_ACC, DP), jnp.float32),
        mesh=mesh,
        compiler_params=pltpu.CompilerParams(use_tc_tiling_on_sc=False),
        scratch_types=[
            pltpu.VMEM((4, BLK), jnp.int32),
            pltpu.VMEM((4, BLK), jnp.int32),
            pltpu.VMEM((4, BLK), jnp.float32),
            pltpu.VMEM((BLK, DP), jnp.float32),
            pltpu.VMEM((BLK, DP), jnp.float32),
            pltpu.VMEM_SHARED((N_ACC, DP), jnp.float32),
            pltpu.SemaphoreType.DMA,
            pltpu.SemaphoreType.DMA,
            pltpu.SemaphoreType.DMA,
            pltpu.SemaphoreType.DMA,
            pltpu.SemaphoreType.DMA,
            pltpu.SemaphoreType.DMA,
            pltpu.SemaphoreType.DMA,
            pltpu.SemaphoreType.DMA,
        ],
    )
    return kern(h_pad, edge_index, edge_weight)


def _post_body(p_ref, o_ref):
    sx = p_ref[0] + p_ref[1]
    s0 = sx[:, 0:1]
    mdot = jnp.sum(sx * sx, axis=1, keepdims=True) - 2.0 * s0 * s0
    coef = 1.0 / jnp.sqrt(jnp.maximum(jnp.abs(mdot), EPS))
    h = coef * sx
    h0 = h[:, 0:1]
    y = h[:, 1:D]
    y_norm = jnp.maximum(
        jnp.sqrt(jnp.sum(y * y, axis=1, keepdims=True)), MIN_NORM)
    theta = jnp.maximum(h0, 1.0 + EPS)
    xt = jnp.maximum(_arcosh(theta) * y / y_norm, 0.0)
    v_norm = jnp.maximum(
        jnp.sqrt(jnp.sum(xt * xt, axis=1, keepdims=True)), MIN_NORM)
    sp = _sinh(v_norm) * xt / v_norm
    t = jnp.sqrt(jnp.maximum(
        1.0 + jnp.sum(sp * sp, axis=1, keepdims=True), EPS))
    o_ref[...] = sp / (t + 1.0)


def _postprocess(partials):
    grid = N_NODES // TC_BLK
    return pl.pallas_call(
        _post_body,
        grid=(grid,),
        in_specs=[pl.BlockSpec((NC, TC_BLK, DP), lambda i: (0, i, 0))],
        out_specs=pl.BlockSpec((TC_BLK, 128), lambda i: (i, 0)),
        out_shape=jax.ShapeDtypeStruct((N_NODES, 128), jnp.float32),
    )(partials)


def kernel(x, edge_index, edge_weight, weight, dec_bias):
    h_pad = _lorentz_linear(x, weight)

    partials = _sc_segment_sum(h_pad, edge_index, edge_weight)
    return _postprocess(partials)
